# baseline (speedup 1.0000x reference)
import functools

import jax
import jax.numpy as jnp
from jax import lax
from jax.experimental import pallas as pl
from jax.experimental.pallas import tpu as pltpu
from jax.experimental.pallas import tpu_sc as plsc

N = 10000
E = 320000
NPAD = 10240
NC = 2
NS = 16
NRP = NPAD // NS
EP = E // NS
CHD = 2000

_mesh = plsc.VectorSubcoreMesh(
    core_axis_name="c", subcore_axis_name="s", num_cores=NC, num_subcores=NS
)
_sc_params = pltpu.CompilerParams(
    needs_layout_passes=False, use_tc_tiling_on_sc=False
)


def _rsqrt16(v):
    xh = v * 0.5
    yi = 0x5F3759DF - (plsc.bitcast(v, jnp.int32) >> 1)
    y = plsc.bitcast(yi, jnp.float32)
    for _ in range(4):
        y = y * (1.5 - xh * y * y)
    return y


@functools.partial(
    pl.kernel,
    out_type=[
        jax.ShapeDtypeStruct((NC, NPAD, 64), jnp.float32),
        jax.ShapeDtypeStruct((NPAD,), jnp.float32),
    ],
    mesh=_mesh,
    compiler_params=_sc_params,
    scratch_types=[
        pltpu.VMEM((10000,), jnp.int32),
        pltpu.VMEM((10000,), jnp.float32),
        [pltpu.VMEM((400,), jnp.int32)] * 2,
        [pltpu.VMEM((400, 64), jnp.float32)] * 2,
        [pltpu.VMEM((400,), jnp.float32)] * 2,
        [pltpu.VMEM((CHD,), jnp.int32)] * 2,
        [pltpu.VMEM((CHD,), jnp.float32)] * 2,
        pltpu.VMEM_SHARED((NPAD,), jnp.float32),
        pltpu.VMEM_SHARED((NPAD,), jnp.float32),
        pltpu.VMEM_SHARED((NPAD, 64), jnp.float32),
        pltpu.SemaphoreType.DMA,
        [pltpu.SemaphoreType.DMA] * 2,
        [pltpu.SemaphoreType.DMA] * 2,
        [pltpu.SemaphoreType.DMA] * 2,
        [pltpu.SemaphoreType.DMA] * 2,
        [pltpu.SemaphoreType.DMA] * 2,
        [pltpu.SemaphoreType.DMA] * 2,
    ],
)
def _sca(ei_h, ew_h, h_h, zd_h, z_h, acc_h, dinv_h,
         row_i, ew_i, col_b, rows_b, dr_b, dgc_b, dgw_b,
         deg_sh, dinv_sh, acc_sh,
         isem, csem, gsem, dsem, ssem, dgisem, dgssem):
    c = lax.axis_index("c")
    s = lax.axis_index("s")
    CH, IB = 400, 10000
    nchunks = EP // CH
    cpb = IB // CH
    base = s * EP

    def colcopy(k, slot):
        return pltpu.async_copy(
            ei_h.at[1, pl.ds(pl.multiple_of(base + k * CH, 8), CH)],
            col_b[slot], csem[slot])

    d_row = pltpu.async_copy(ei_h.at[0, pl.ds(base, IB)], row_i, isem)
    d_ew = pltpu.async_copy(ew_h.at[pl.ds(base, IB)], ew_i, isem)
    cd = [colcopy(0, 0), colcopy(1, 1)]
    pltpu.sync_copy(zd_h, deg_sh.at[pl.ds(s * NRP, NRP)])
    pltpu.sync_copy(z_h, acc_sh.at[pl.ds(s * NRP, NRP)])
    plsc.subcore_barrier()

    ndeg = EP // CHD

    def degcopy(kk, sl):
        off = pl.multiple_of(base + kk * CHD, 8)
        a = pltpu.async_copy(ei_h.at[1, pl.ds(off, CHD)], dgc_b[sl],
                             dgisem[sl])
        b = pltpu.async_copy(ew_h.at[pl.ds(off, CHD)], dgw_b[sl], dgisem[sl])
        return a, b

    dd = [degcopy(0, 0), None]
    dsc = [None, None]
    for kk in range(ndeg):
        sl = kk % 2
        nsl = (kk + 1) % 2
        dd[sl][0].wait()
        dd[sl][1].wait()
        if kk >= 1:
            dsc[nsl].wait()
        if kk + 1 < ndeg:
            dd[nsl] = degcopy(kk + 1, nsl)
        dsc[sl] = pltpu.async_copy(dgw_b[sl], deg_sh.at[dgc_b[sl]],
                                   dgssem[sl], add=True)
    dsc[(ndeg - 1) % 2].wait()
    plsc.subcore_barrier()

    pltpu.sync_copy(deg_sh.at[pl.ds(s * NRP, NRP)],
                    dgw_b[0].at[pl.ds(0, NRP)])

    def dinv_body(i, carry):
        v = dgw_b[0][pl.ds(i * 16, 16)]
        dgw_b[1][pl.ds(i * 16, 16)] = _rsqrt16(v + 1.0)
        return carry

    lax.fori_loop(0, NRP // 16, dinv_body, 0)
    pltpu.sync_copy(dgw_b[1].at[pl.ds(0, NRP)],
                    dinv_sh.at[pl.ds(s * NRP, NRP)])

    @pl.when(c == 0)
    def _():
        pltpu.sync_copy(dgw_b[1].at[pl.ds(0, NRP)],
                        dinv_h.at[pl.ds(s * NRP, NRP)])

    plsc.subcore_barrier()

    d_row.wait()
    d_ew.wait()

    g = [None, None]
    dg = [None, None]
    sc = [None, None]

    def start_gather(k, slot):
        idx = row_i.at[pl.ds((k * CH) % IB, CH)]
        dg[slot] = pltpu.async_copy(dinv_sh.at[idx], dr_b[slot], dsem[slot])
        return pltpu.async_copy(h_h.at[c].at[idx], rows_b[slot], gsem[slot])

    g[0] = start_gather(0, 0)
    for k in range(nchunks):
        slot = k % 2
        nslot = (k + 1) % 2
        g[slot].wait()
        dg[slot].wait()
        if (k + 1) % cpb == 0 and k + 1 < nchunks:
            pltpu.sync_copy(ei_h.at[0, pl.ds(base + (k + 1) * CH, IB)],
                            row_i)
        if k >= 1:
            sc[nslot].wait()
        if k + 1 < nchunks:
            g[nslot] = start_gather(k + 1, nslot)
            if k >= 1:
                cd[nslot] = colcopy(k + 1, nslot)
        if k % cpb == 0 and k > 0:
            pltpu.sync_copy(ew_h.at[pl.ds(base + k * CH, IB)], ew_i)

        rv = rows_b[slot]
        dr = dr_b[slot]
        loff = (k * CH) % IB

        def we_body(i, carry):
            dr[pl.ds(i * 16, 16)] = (
                dr[pl.ds(i * 16, 16)] * ew_i[pl.ds(loff + i * 16, 16)]
            )
            return carry

        lax.fori_loop(0, CH // 16, we_body, 0)

        @plsc.parallel_loop(0, CH, 1, unroll=4)
        def _(e):
            wv = plsc.load_gather(dr, [jnp.full((16,), e, jnp.int32)])
            for fb in range(4):
                rv[e, pl.ds(fb * 16, 16)] = rv[e, pl.ds(fb * 16, 16)] * wv

        cd[slot].wait()
        sc[slot] = pltpu.async_copy(rv, acc_sh.at[col_b[slot]],
                                    ssem[slot], add=True)
    sc[(nchunks - 1) % 2].wait()
    plsc.subcore_barrier()
    pltpu.sync_copy(acc_sh.at[pl.ds(s * NRP, NRP)],
                    acc_h.at[c, pl.ds(s * NRP, NRP)])


def _make_agg(D, CH, IB):
    nchunks = EP // CH
    cpb = IB // CH

    @functools.partial(
        pl.kernel,
        out_type=jax.ShapeDtypeStruct((NC, NPAD, D), jnp.float32),
        mesh=_mesh,
        compiler_params=_sc_params,
        scratch_types=[
            pltpu.VMEM((IB,), jnp.int32),
            pltpu.VMEM((IB,), jnp.float32),
            [pltpu.VMEM((CH,), jnp.int32)] * 2,
            [pltpu.VMEM((CH, D), jnp.float32)] * 2,
            pltpu.VMEM_SHARED((NPAD, D), jnp.float32),
            pltpu.SemaphoreType.DMA,
            [pltpu.SemaphoreType.DMA] * 2,
            [pltpu.SemaphoreType.DMA] * 2,
            [pltpu.SemaphoreType.DMA] * 2,
        ],
    )
    def agg(ei_h, ew_h, p_h, z_h, acc_h, row_i, ew_i, col_b,
            rows_b, acc_sh, isem, csem, gsem, ssem):
        c = lax.axis_index("c")
        s = lax.axis_index("s")
        base = s * EP

        def colcopy(k, slot):
            return pltpu.async_copy(
                ei_h.at[1, pl.ds(pl.multiple_of(base + k * CH, 8), CH)],
                col_b[slot], csem[slot])

        d_row = pltpu.async_copy(ei_h.at[0, pl.ds(base, IB)], row_i, isem)
        d_ew = pltpu.async_copy(ew_h.at[pl.ds(base, IB)], ew_i, isem)
        cd = [colcopy(0, 0), colcopy(1, 1)]
        pltpu.sync_copy(z_h, acc_sh.at[pl.ds(s * NRP, NRP)])
        plsc.subcore_barrier()
        d_row.wait()
        d_ew.wait()

        g = [None, None]
        sc = [None, None]

        def start_gather(k, slot):
            idx = row_i.at[pl.ds((k * CH) % IB, CH)]
            return pltpu.async_copy(p_h.at[c].at[idx], rows_b[slot],
                                    gsem[slot])

        g[0] = start_gather(0, 0)
        for k in range(nchunks):
            slot = k % 2
            nslot = (k + 1) % 2
            g[slot].wait()
            if (k + 1) % cpb == 0 and k + 1 < nchunks:
                pltpu.sync_copy(ei_h.at[0, pl.ds(base + (k + 1) * CH, IB)],
                                row_i)
            if k >= 1:
                sc[nslot].wait()
            if k + 1 < nchunks:
                g[nslot] = start_gather(k + 1, nslot)
                if k >= 1:
                    cd[nslot] = colcopy(k + 1, nslot)
            if k % cpb == 0 and k > 0:
                pltpu.sync_copy(ew_h.at[pl.ds(base + k * CH, IB)], ew_i)

            rv = rows_b[slot]
            loff = (k * CH) % IB

            @plsc.parallel_loop(0, CH, 1, unroll=8)
            def _(e):
                wv = plsc.load_gather(
                    ew_i, [jnp.full((16,), loff + e, jnp.int32)])
                for fb in range(D // 16):
                    rv[e, pl.ds(fb * 16, 16)] = rv[e, pl.ds(fb * 16, 16)] * wv

            cd[slot].wait()
            sc[slot] = pltpu.async_copy(rv, acc_sh.at[col_b[slot]],
                                        ssem[slot], add=True)
        sc[(nchunks - 1) % 2].wait()
        plsc.subcore_barrier()
        pltpu.sync_copy(acc_sh.at[pl.ds(s * NRP, NRP)],
                        acc_h.at[c, pl.ds(s * NRP, NRP)])

    return agg


_agg32 = _make_agg(32, 1000, 20000)


BN = 10240
GRID = NPAD // BN


def _tc1_body(x_ref, w1_ref, h_ref):
    h = jnp.dot(x_ref[...], w1_ref[...], preferred_element_type=jnp.float32)
    h_ref[0] = h[:, :64]
    h_ref[1] = h[:, 64:]


def _tc1(x, W1):
    return pl.pallas_call(
        _tc1_body,
        grid=(GRID,),
        in_specs=[
            pl.BlockSpec((BN, 128), lambda i: (i, 0)),
            pl.BlockSpec((128, 128), lambda i: (0, 0)),
        ],
        out_specs=pl.BlockSpec((2, BN, 64), lambda i: (0, i, 0)),
        out_shape=jax.ShapeDtypeStruct((2, N, 64), jnp.float32),
    )(x, W1)


def _tc2_body(a_ref, h_ref, dinv_ref, b1_ref, w2_ref, q_ref):
    dinv = dinv_ref[...]
    o = jnp.concatenate(
        [a_ref[0] + dinv[:, None] * h_ref[0],
         a_ref[1] + dinv[:, None] * h_ref[1]], axis=1)
    h1 = jax.nn.relu(o * dinv[:, None] + b1_ref[...][None, :])
    q = jnp.dot(h1, w2_ref[...], preferred_element_type=jnp.float32)
    q = q * dinv[:, None]
    q_ref[0] = q[:, :32]
    q_ref[1] = q[:, 32:]


def _tc2(a, h, dinv, b1, W2):
    return pl.pallas_call(
        _tc2_body,
        grid=(GRID,),
        in_specs=[
            pl.BlockSpec((2, BN, 64), lambda i: (0, i, 0)),
            pl.BlockSpec((2, BN, 64), lambda i: (0, i, 0)),
            pl.BlockSpec((BN,), lambda i: (i,)),
            pl.BlockSpec((128,), lambda i: (0,)),
            pl.BlockSpec((128, 64), lambda i: (0, 0)),
        ],
        out_specs=pl.BlockSpec((2, BN, 32), lambda i: (0, i, 0)),
        out_shape=jax.ShapeDtypeStruct((2, N, 32), jnp.float32),
    )(a, h, dinv, b1, W2)


def _tc3_body(a_ref, q_ref, dinv_ref, b2_ref, out_ref):
    acc = jnp.concatenate(
        [a_ref[0] + q_ref[0], a_ref[1] + q_ref[1]], axis=1)
    o = acc * dinv_ref[...][:, None] + b2_ref[...][None, :]
    m = jnp.max(o, axis=1, keepdims=True)
    z = o - m
    out_ref[...] = z - jnp.log(jnp.sum(jnp.exp(z), axis=1, keepdims=True))


def _tc3(a, q, dinv, b2):
    return pl.pallas_call(
        _tc3_body,
        grid=(GRID,),
        in_specs=[
            pl.BlockSpec((2, BN, 32), lambda i: (0, i, 0)),
            pl.BlockSpec((2, BN, 32), lambda i: (0, i, 0)),
            pl.BlockSpec((BN,), lambda i: (i,)),
            pl.BlockSpec((64,), lambda i: (0,)),
        ],
        out_specs=pl.BlockSpec((BN, 64), lambda i: (i, 0)),
        out_shape=jax.ShapeDtypeStruct((N, 64), jnp.float32),
    )(a, q, dinv, b2)


def kernel(x, edge_index, edge_weight, W1, b1, W2, b2):
    ei = edge_index.astype(jnp.int32)
    ew = edge_weight.astype(jnp.float32)

    zdeg = jnp.zeros((NRP,), jnp.float32)
    z64 = jnp.zeros((NRP, 64), jnp.float32)
    z32 = jnp.zeros((NRP, 32), jnp.float32)

    h = _tc1(x, W1)
    acc1, dinv = _sca(ei, ew, h, zdeg, z64)
    q = _tc2(acc1, h, dinv, b1, W2)
    acc2 = _agg32(ei, ew, q, z32)
    return _tc3(acc2, q, dinv, b2)


# speedup vs baseline: 1.0270x; 1.0270x over previous
import functools

import jax
import jax.numpy as jnp
from jax import lax
from jax.experimental import pallas as pl
from jax.experimental.pallas import tpu as pltpu
from jax.experimental.pallas import tpu_sc as plsc

N = 10000
E = 320000
NPAD = 10240
NC = 2
NS = 16
NRP = NPAD // NS
EP = E // NS
CHD = 2000

_mesh = plsc.VectorSubcoreMesh(
    core_axis_name="c", subcore_axis_name="s", num_cores=NC, num_subcores=NS
)
_sc_params = pltpu.CompilerParams(
    needs_layout_passes=False, use_tc_tiling_on_sc=False
)


def _rsqrt16(v):
    xh = v * 0.5
    yi = 0x5F3759DF - (plsc.bitcast(v, jnp.int32) >> 1)
    y = plsc.bitcast(yi, jnp.float32)
    for _ in range(4):
        y = y * (1.5 - xh * y * y)
    return y


@functools.partial(
    pl.kernel,
    out_type=[
        jax.ShapeDtypeStruct((NC, NPAD, 64), jnp.float32),
        jax.ShapeDtypeStruct((NPAD,), jnp.float32),
    ],
    mesh=_mesh,
    compiler_params=_sc_params,
    scratch_types=[
        pltpu.VMEM((10000,), jnp.int32),
        pltpu.VMEM((10000,), jnp.float32),
        [pltpu.VMEM((400,), jnp.int32)] * 2,
        [pltpu.VMEM((400, 64), jnp.float32)] * 2,
        [pltpu.VMEM((400,), jnp.float32)] * 2,
        [pltpu.VMEM((CHD,), jnp.int32)] * 2,
        [pltpu.VMEM((CHD,), jnp.float32)] * 2,
        pltpu.VMEM_SHARED((NPAD,), jnp.float32),
        pltpu.VMEM_SHARED((NPAD,), jnp.float32),
        pltpu.VMEM_SHARED((NPAD, 64), jnp.float32),
        pltpu.SemaphoreType.DMA,
        [pltpu.SemaphoreType.DMA] * 2,
        [pltpu.SemaphoreType.DMA] * 2,
        [pltpu.SemaphoreType.DMA] * 2,
        [pltpu.SemaphoreType.DMA] * 2,
        [pltpu.SemaphoreType.DMA] * 2,
        [pltpu.SemaphoreType.DMA] * 2,
    ],
)
def _sca(ei_h, ew_h, h_h, zd_h, z_h, acc_h, dinv_h,
         row_i, ew_i, col_b, rows_b, dr_b, dgc_b, dgw_b,
         deg_sh, dinv_sh, acc_sh,
         isem, csem, gsem, dsem, ssem, dgisem, dgssem):
    c = lax.axis_index("c")
    s = lax.axis_index("s")
    CH, IB = 400, 10000
    nchunks = EP // CH
    cpb = IB // CH
    base = s * EP

    def colcopy(k, slot):
        return pltpu.async_copy(
            ei_h.at[1, pl.ds(pl.multiple_of(base + k * CH, 8), CH)],
            col_b[slot], csem[slot])

    d_row = pltpu.async_copy(ei_h.at[0, pl.ds(base, IB)], row_i, isem)
    d_ew = pltpu.async_copy(ew_h.at[pl.ds(base, IB)], ew_i, isem)
    cd = [colcopy(0, 0), colcopy(1, 1)]
    pltpu.sync_copy(zd_h, deg_sh.at[pl.ds(s * NRP, NRP)])
    pltpu.sync_copy(z_h, acc_sh.at[pl.ds(s * NRP, NRP)])
    plsc.subcore_barrier()

    ndeg = EP // CHD

    def degcopy(kk, sl):
        off = pl.multiple_of(base + kk * CHD, 8)
        a = pltpu.async_copy(ei_h.at[1, pl.ds(off, CHD)], dgc_b[sl],
                             dgisem[sl])
        b = pltpu.async_copy(ew_h.at[pl.ds(off, CHD)], dgw_b[sl], dgisem[sl])
        return a, b

    dd = [degcopy(0, 0), None]
    dsc = [None, None]
    for kk in range(ndeg):
        sl = kk % 2
        nsl = (kk + 1) % 2
        dd[sl][0].wait()
        dd[sl][1].wait()
        if kk >= 1:
            dsc[nsl].wait()
        if kk + 1 < ndeg:
            dd[nsl] = degcopy(kk + 1, nsl)
        dsc[sl] = pltpu.async_copy(dgw_b[sl], deg_sh.at[dgc_b[sl]],
                                   dgssem[sl], add=True)
    dsc[(ndeg - 1) % 2].wait()
    plsc.subcore_barrier()

    pltpu.sync_copy(deg_sh.at[pl.ds(s * NRP, NRP)],
                    dgw_b[0].at[pl.ds(0, NRP)])

    def dinv_body(i, carry):
        v = dgw_b[0][pl.ds(i * 16, 16)]
        dgw_b[1][pl.ds(i * 16, 16)] = _rsqrt16(v + 1.0)
        return carry

    lax.fori_loop(0, NRP // 16, dinv_body, 0)
    pltpu.sync_copy(dgw_b[1].at[pl.ds(0, NRP)],
                    dinv_sh.at[pl.ds(s * NRP, NRP)])

    @pl.when(c == 0)
    def _():
        pltpu.sync_copy(dgw_b[1].at[pl.ds(0, NRP)],
                        dinv_h.at[pl.ds(s * NRP, NRP)])

    plsc.subcore_barrier()

    d_row.wait()
    d_ew.wait()

    g = [None, None]
    dg = [None, None]
    sc = [None, None]

    def start_gather(k, slot):
        idx = row_i.at[pl.ds((k * CH) % IB, CH)]
        dg[slot] = pltpu.async_copy(dinv_sh.at[idx], dr_b[slot], dsem[slot])
        return pltpu.async_copy(h_h.at[c].at[idx], rows_b[slot], gsem[slot])

    g[0] = start_gather(0, 0)
    for k in range(nchunks):
        slot = k % 2
        nslot = (k + 1) % 2
        g[slot].wait()
        dg[slot].wait()
        if (k + 1) % cpb == 0 and k + 1 < nchunks:
            pltpu.sync_copy(ei_h.at[0, pl.ds(base + (k + 1) * CH, IB)],
                            row_i)
        if k >= 1:
            sc[nslot].wait()
        if k + 1 < nchunks:
            g[nslot] = start_gather(k + 1, nslot)
            if k >= 1:
                cd[nslot] = colcopy(k + 1, nslot)
        if k % cpb == 0 and k > 0:
            pltpu.sync_copy(ew_h.at[pl.ds(base + k * CH, IB)], ew_i)

        rv = rows_b[slot]
        dr = dr_b[slot]
        loff = (k * CH) % IB

        def we_body(i, carry):
            dr[pl.ds(i * 16, 16)] = (
                dr[pl.ds(i * 16, 16)] * ew_i[pl.ds(loff + i * 16, 16)]
            )
            return carry

        lax.fori_loop(0, CH // 16, we_body, 0)

        @plsc.parallel_loop(0, CH, 1, unroll=4)
        def _(e):
            wv = plsc.load_gather(dr, [jnp.full((16,), e, jnp.int32)])
            for fb in range(4):
                rv[e, pl.ds(fb * 16, 16)] = rv[e, pl.ds(fb * 16, 16)] * wv

        cd[slot].wait()
        sc[slot] = pltpu.async_copy(rv, acc_sh.at[col_b[slot]],
                                    ssem[slot], add=True)
    sc[(nchunks - 1) % 2].wait()
    plsc.subcore_barrier()
    pltpu.sync_copy(acc_sh.at[pl.ds(s * NRP, NRP)],
                    acc_h.at[c, pl.ds(s * NRP, NRP)])


def _make_agg(D, CH, IB):
    nchunks = EP // CH
    cpb = IB // CH

    @functools.partial(
        pl.kernel,
        out_type=jax.ShapeDtypeStruct((NC, NPAD, D), jnp.float32),
        mesh=_mesh,
        compiler_params=_sc_params,
        scratch_types=[
            pltpu.VMEM((IB,), jnp.int32),
            pltpu.VMEM((IB,), jnp.float32),
            [pltpu.VMEM((CH,), jnp.int32)] * 2,
            [pltpu.VMEM((CH, D), jnp.float32)] * 2,
            pltpu.VMEM_SHARED((NPAD, D), jnp.float32),
            pltpu.SemaphoreType.DMA,
            [pltpu.SemaphoreType.DMA] * 2,
            [pltpu.SemaphoreType.DMA] * 2,
            [pltpu.SemaphoreType.DMA] * 2,
        ],
    )
    def agg(ei_h, ew_h, p_h, z_h, acc_h, row_i, ew_i, col_b,
            rows_b, acc_sh, isem, csem, gsem, ssem):
        c = lax.axis_index("c")
        s = lax.axis_index("s")
        base = s * EP

        def colcopy(k, slot):
            return pltpu.async_copy(
                ei_h.at[1, pl.ds(pl.multiple_of(base + k * CH, 8), CH)],
                col_b[slot], csem[slot])

        d_row = pltpu.async_copy(ei_h.at[0, pl.ds(base, IB)], row_i, isem)
        d_ew = pltpu.async_copy(ew_h.at[pl.ds(base, IB)], ew_i, isem)
        cd = [colcopy(0, 0), colcopy(1, 1)]
        pltpu.sync_copy(z_h, acc_sh.at[pl.ds(s * NRP, NRP)])
        plsc.subcore_barrier()
        d_row.wait()
        d_ew.wait()

        g = [None, None]
        sc = [None, None]

        def start_gather(k, slot):
            idx = row_i.at[pl.ds((k * CH) % IB, CH)]
            return pltpu.async_copy(p_h.at[c].at[idx], rows_b[slot],
                                    gsem[slot])

        g[0] = start_gather(0, 0)
        for k in range(nchunks):
            slot = k % 2
            nslot = (k + 1) % 2
            g[slot].wait()
            if (k + 1) % cpb == 0 and k + 1 < nchunks:
                pltpu.sync_copy(ei_h.at[0, pl.ds(base + (k + 1) * CH, IB)],
                                row_i)
            if k >= 1:
                sc[nslot].wait()
            if k + 1 < nchunks:
                g[nslot] = start_gather(k + 1, nslot)
                if k >= 1:
                    cd[nslot] = colcopy(k + 1, nslot)
            if k % cpb == 0 and k > 0:
                pltpu.sync_copy(ew_h.at[pl.ds(base + k * CH, IB)], ew_i)

            rv = rows_b[slot]
            loff = (k * CH) % IB

            @plsc.parallel_loop(0, CH, 1, unroll=8)
            def _(e):
                wv = plsc.load_gather(
                    ew_i, [jnp.full((16,), loff + e, jnp.int32)])
                for fb in range(D // 16):
                    rv[e, pl.ds(fb * 16, 16)] = rv[e, pl.ds(fb * 16, 16)] * wv

            cd[slot].wait()
            sc[slot] = pltpu.async_copy(rv, acc_sh.at[col_b[slot]],
                                        ssem[slot], add=True)
        sc[(nchunks - 1) % 2].wait()
        plsc.subcore_barrier()
        pltpu.sync_copy(acc_sh.at[pl.ds(s * NRP, NRP)],
                        acc_h.at[c, pl.ds(s * NRP, NRP)])

    return agg


_agg32 = _make_agg(32, 1000, 20000)


BN = 5120
GRID = NPAD // BN


def _tc1_body(x_ref, w1_ref, h_ref):
    h = jnp.dot(x_ref[...], w1_ref[...], preferred_element_type=jnp.float32)
    h_ref[0] = h[:, :64]
    h_ref[1] = h[:, 64:]


def _tc1(x, W1):
    return pl.pallas_call(
        _tc1_body,
        grid=(GRID,),
        in_specs=[
            pl.BlockSpec((BN, 128), lambda i: (i, 0)),
            pl.BlockSpec((128, 128), lambda i: (0, 0)),
        ],
        out_specs=pl.BlockSpec((2, BN, 64), lambda i: (0, i, 0)),
        out_shape=jax.ShapeDtypeStruct((2, N, 64), jnp.float32),
    )(x, W1)


def _tc2_body(a_ref, h_ref, dinv_ref, b1_ref, w2_ref, q_ref):
    dinv = dinv_ref[...]
    o = jnp.concatenate(
        [a_ref[0] + dinv[:, None] * h_ref[0],
         a_ref[1] + dinv[:, None] * h_ref[1]], axis=1)
    h1 = jax.nn.relu(o * dinv[:, None] + b1_ref[...][None, :])
    q = jnp.dot(h1, w2_ref[...], preferred_element_type=jnp.float32)
    q = q * dinv[:, None]
    q_ref[0] = q[:, :32]
    q_ref[1] = q[:, 32:]


def _tc2(a, h, dinv, b1, W2):
    return pl.pallas_call(
        _tc2_body,
        grid=(GRID,),
        in_specs=[
            pl.BlockSpec((2, BN, 64), lambda i: (0, i, 0)),
            pl.BlockSpec((2, BN, 64), lambda i: (0, i, 0)),
            pl.BlockSpec((BN,), lambda i: (i,)),
            pl.BlockSpec((128,), lambda i: (0,)),
            pl.BlockSpec((128, 64), lambda i: (0, 0)),
        ],
        out_specs=pl.BlockSpec((2, BN, 32), lambda i: (0, i, 0)),
        out_shape=jax.ShapeDtypeStruct((2, N, 32), jnp.float32),
    )(a, h, dinv, b1, W2)


def _tc3_body(a_ref, q_ref, dinv_ref, b2_ref, out_ref):
    acc = jnp.concatenate(
        [a_ref[0] + q_ref[0], a_ref[1] + q_ref[1]], axis=1)
    o = acc * dinv_ref[...][:, None] + b2_ref[...][None, :]
    m = jnp.max(o, axis=1, keepdims=True)
    z = o - m
    out_ref[...] = z - jnp.log(jnp.sum(jnp.exp(z), axis=1, keepdims=True))


def _tc3(a, q, dinv, b2):
    return pl.pallas_call(
        _tc3_body,
        grid=(GRID,),
        in_specs=[
            pl.BlockSpec((2, BN, 32), lambda i: (0, i, 0)),
            pl.BlockSpec((2, BN, 32), lambda i: (0, i, 0)),
            pl.BlockSpec((BN,), lambda i: (i,)),
            pl.BlockSpec((64,), lambda i: (0,)),
        ],
        out_specs=pl.BlockSpec((BN, 64), lambda i: (i, 0)),
        out_shape=jax.ShapeDtypeStruct((N, 64), jnp.float32),
    )(a, q, dinv, b2)


def kernel(x, edge_index, edge_weight, W1, b1, W2, b2):
    ei = edge_index.astype(jnp.int32)
    ew = edge_weight.astype(jnp.float32)

    zdeg = jnp.zeros((NRP,), jnp.float32)
    z64 = jnp.zeros((NRP, 64), jnp.float32)
    z32 = jnp.zeros((NRP, 32), jnp.float32)

    h = _tc1(x, W1)
    acc1, dinv = _sca(ei, ew, h, zdeg, z64)
    q = _tc2(acc1, h, dinv, b1, W2)
    acc2 = _agg32(ei, ew, q, z32)
    return _tc3(acc2, q, dinv, b2)
